# Initial kernel scaffold; baseline (speedup 1.0000x reference)
import jax
import jax.numpy as jnp
from jax.experimental import pallas as pl


def kernel(samples, table):
    raise NotImplementedError("write your pallas kernel here")


# speedup vs baseline: 3.8098x; 3.8098x over previous
import functools

import jax
import jax.numpy as jnp
from jax import lax
from jax.experimental import pallas as pl
from jax.experimental.pallas import tpu as pltpu
from jax.experimental.pallas import tpu_sc as plsc

BATCH = 4096
SEQ = 200
SEQ_PAD = 256
EMBED = 64
CHUNK = 512
NCHUNK = BATCH // CHUNK
SEQ_SPLITS = ((0, 104), (104, 96))


def _permute_body(s_ref, ps_ref, lrow_ref, rank_ref):
    f32 = jnp.float32
    s = s_ref[...]
    mask = (s > 0).astype(f32)
    lcol = jnp.sum(mask, axis=1, keepdims=True)

    i0 = lax.broadcasted_iota(jnp.int32, (CHUNK, CHUNK), 0)
    i1 = lax.broadcasted_iota(jnp.int32, (CHUNK, CHUNK), 1)
    eye = (i0 == i1).astype(f32)
    for c in range(NCHUNK):
        lc = lcol[c * CHUNK:(c + 1) * CHUNK, :]
        lrow_ref[:, c * CHUNK:(c + 1) * CHUNK] = lax.dot_general(
            lc, eye, (((0,), (0,)), ((), ())))
    lrow = lrow_ref[...]

    jj = lax.broadcasted_iota(jnp.int32, (CHUNK, BATCH), 1)
    ii = lax.broadcasted_iota(jnp.int32, (CHUNK, BATCH), 0)
    for c in range(NCHUNK):
        li = lcol[c * CHUNK:(c + 1) * CHUNK, :]
        gi = ii + c * CHUNK
        gt = (lrow > li).astype(f32)
        tie = ((lrow == li) & (jj < gi)).astype(f32)
        rank_ref[c * CHUNK:(c + 1) * CHUNK, :] = jnp.sum(
            gt + tie, axis=1, keepdims=True)

    jjf = jj.astype(f32)
    ps = jnp.zeros((BATCH, SEQ_PAD), f32)
    dims = (((0,), (0,)), ((), ()))
    for c in range(NCHUNK):
        rc = rank_ref[c * CHUNK:(c + 1) * CHUNK, :]
        match = (rc == jjf).astype(f32)
        sc_rows = s[c * CHUNK:(c + 1) * CHUNK, :]
        pa = lax.dot_general(
            match, (sc_rows >> 16).astype(f32), dims)
        pb = lax.dot_general(
            match, ((sc_rows >> 8) & 255).astype(f32), dims)
        pc = lax.dot_general(
            match, (sc_rows & 255).astype(f32), dims)
        ps = ps + (pa * 65536.0 + pb * 256.0 + pc)
    ps_ref[...] = ps.astype(jnp.int32)


def _tc_permute(samples_padded):
    return pl.pallas_call(
        _permute_body,
        out_shape=jax.ShapeDtypeStruct((BATCH, SEQ_PAD), jnp.int32),
        scratch_shapes=[
            pltpu.VMEM((1, BATCH), jnp.float32),
            pltpu.VMEM((BATCH, 1), jnp.float32),
        ],
    )(samples_padded)


def _sc_gather(ps_flat, table):
    info = plsc.get_sparse_core_info()
    nc, ns = info.num_cores, info.num_subcores
    nw = nc * ns
    rows_per_w = BATCH // nw
    mesh = plsc.VectorSubcoreMesh(core_axis_name="c", subcore_axis_name="s")

    @functools.partial(
        pl.kernel,
        mesh=mesh,
        out_type=jax.ShapeDtypeStruct((BATCH * SEQ, EMBED), jnp.float32),
        scratch_types=[
            pltpu.VMEM((rows_per_w * SEQ_PAD,), jnp.int32),
            pltpu.VMEM((SEQ, EMBED), jnp.float32),
            pltpu.SemaphoreType.DMA,
        ],
        compiler_params=pltpu.CompilerParams(use_tc_tiling_on_sc=False),
    )
    def k(ps_hbm, table_hbm, out_hbm, ids_v, buf, sem):
        wid = lax.axis_index("s") * nc + lax.axis_index("c")
        base = wid * rows_per_w
        pltpu.sync_copy(ps_hbm.at[pl.ds(base * SEQ_PAD, rows_per_w * SEQ_PAD)],
                        ids_v)

        def row(p, carry):
            for off, n in SEQ_SPLITS:
                pltpu.async_copy(
                    table_hbm.at[ids_v.at[pl.ds(p * SEQ_PAD + off, n)]],
                    buf.at[pl.ds(off, n)], sem).wait()
            pltpu.sync_copy(buf, out_hbm.at[pl.ds((base + p) * SEQ, SEQ)])
            return carry

        lax.fori_loop(0, rows_per_w, row, 0)

    return k(ps_flat, table)


def kernel(samples, table):
    s32 = samples.astype(jnp.int32)
    sp = jnp.pad(s32, ((0, 0), (0, SEQ_PAD - SEQ)))
    ps = _tc_permute(sp)
    out = _sc_gather(ps.reshape(BATCH * SEQ_PAD), table)
    return out.reshape(BATCH, SEQ, EMBED)


# speedup vs baseline: 4.6083x; 1.2096x over previous
import functools

import jax
import jax.numpy as jnp
from jax import lax
from jax.experimental import pallas as pl
from jax.experimental.pallas import tpu as pltpu
from jax.experimental.pallas import tpu_sc as plsc

BATCH = 4096
SEQ = 200
SEQ_PAD = 256
EMBED = 64
CHUNK = 512
NCHUNK = BATCH // CHUNK
SEQ_SPLITS = ((0, 104), (104, 96))


def _permute_body(s_ref, ps_ref, lrow_ref, rank_ref):
    f32 = jnp.float32
    s = s_ref[...]
    mask = (s > 0).astype(f32)
    lcol = jnp.sum(mask, axis=1, keepdims=True)

    i0 = lax.broadcasted_iota(jnp.int32, (CHUNK, CHUNK), 0)
    i1 = lax.broadcasted_iota(jnp.int32, (CHUNK, CHUNK), 1)
    eye = (i0 == i1).astype(f32)
    for c in range(NCHUNK):
        lc = lcol[c * CHUNK:(c + 1) * CHUNK, :]
        lrow_ref[:, c * CHUNK:(c + 1) * CHUNK] = lax.dot_general(
            lc, eye, (((0,), (0,)), ((), ())))
    lrow = lrow_ref[...]

    jj = lax.broadcasted_iota(jnp.int32, (CHUNK, BATCH), 1)
    ii = lax.broadcasted_iota(jnp.int32, (CHUNK, BATCH), 0)
    for c in range(NCHUNK):
        li = lcol[c * CHUNK:(c + 1) * CHUNK, :]
        gi = ii + c * CHUNK
        gt = (lrow > li).astype(f32)
        tie = ((lrow == li) & (jj < gi)).astype(f32)
        rank_ref[c * CHUNK:(c + 1) * CHUNK, :] = jnp.sum(
            gt + tie, axis=1, keepdims=True)

    jjf = jj.astype(f32)
    ps = jnp.zeros((BATCH, SEQ_PAD), f32)
    dims = (((0,), (0,)), ((), ()))
    for c in range(NCHUNK):
        rc = rank_ref[c * CHUNK:(c + 1) * CHUNK, :]
        match = (rc == jjf).astype(f32)
        sc_rows = s[c * CHUNK:(c + 1) * CHUNK, :]
        pa = lax.dot_general(
            match, (sc_rows >> 16).astype(f32), dims)
        pb = lax.dot_general(
            match, ((sc_rows >> 8) & 255).astype(f32), dims)
        pc = lax.dot_general(
            match, (sc_rows & 255).astype(f32), dims)
        ps = ps + (pa * 65536.0 + pb * 256.0 + pc)
    ps_ref[...] = ps[:, :SEQ].astype(jnp.int32)


def _tc_permute(samples_padded):
    return pl.pallas_call(
        _permute_body,
        out_shape=jax.ShapeDtypeStruct((BATCH, SEQ), jnp.int32),
        scratch_shapes=[
            pltpu.VMEM((1, BATCH), jnp.float32),
            pltpu.VMEM((BATCH, 1), jnp.float32),
        ],
    )(samples_padded)


GCHUNK = 128
CPS = 4
STAGE = GCHUNK * CPS


def _sc_gather(ps_flat, table):
    info = plsc.get_sparse_core_info()
    nc, ns = info.num_cores, info.num_subcores
    nw = nc * ns
    tpw = BATCH * SEQ // nw
    nstage = tpw // STAGE
    mesh = plsc.VectorSubcoreMesh(core_axis_name="c", subcore_axis_name="s")

    @functools.partial(
        pl.kernel,
        mesh=mesh,
        out_type=jax.ShapeDtypeStruct((BATCH * SEQ, EMBED), jnp.float32),
        scratch_types=[
            pltpu.VMEM((tpw,), jnp.int32),
            pltpu.VMEM((STAGE, EMBED), jnp.float32),
            pltpu.VMEM((STAGE, EMBED), jnp.float32),
            pltpu.SemaphoreType.DMA,
            pltpu.SemaphoreType.DMA,
            pltpu.SemaphoreType.DMA,
            pltpu.SemaphoreType.DMA,
        ],
        compiler_params=pltpu.CompilerParams(use_tc_tiling_on_sc=False),
    )
    def k(ps_hbm, table_hbm, out_hbm, ids_v, s0, s1, g0, g1, w0, w1):
        wid = lax.axis_index("s") * nc + lax.axis_index("c")
        t0 = wid * tpw
        pltpu.sync_copy(ps_hbm.at[pl.ds(t0, tpw)], ids_v)

        def issue_gathers(r, buf, gsem):
            for m in range(CPS):
                pltpu.async_copy(
                    table_hbm.at[ids_v.at[pl.ds(r * STAGE + m * GCHUNK, GCHUNK)]],
                    buf.at[pl.ds(m * GCHUNK, GCHUNK)], gsem)

        def wait_gathers(buf, gsem):
            for m in range(CPS):
                pltpu.make_async_copy(
                    table_hbm.at[ids_v.at[pl.ds(m * GCHUNK, GCHUNK)]],
                    buf.at[pl.ds(m * GCHUNK, GCHUNK)], gsem).wait()

        def issue_write(r, buf, wsem):
            pltpu.async_copy(buf, out_hbm.at[pl.ds(t0 + r * STAGE, STAGE)], wsem)

        def wait_write(r, buf, wsem):
            pltpu.make_async_copy(
                buf, out_hbm.at[pl.ds(t0 + r * STAGE, STAGE)], wsem).wait()

        issue_gathers(0, s0, g0)
        issue_gathers(1, s1, g1)

        def body(kk, carry):
            r = 2 * kk
            wait_gathers(s0, g0)
            issue_write(r, s0, w0)
            wait_gathers(s1, g1)
            issue_write(r + 1, s1, w1)

            @pl.when(r + 2 < nstage)
            def _():
                wait_write(r, s0, w0)
                issue_gathers(r + 2, s0, g0)

            @pl.when(r + 3 < nstage)
            def _():
                wait_write(r + 1, s1, w1)
                issue_gathers(r + 3, s1, g1)

            return carry

        lax.fori_loop(0, nstage // 2, body, 0)
        wait_write(nstage - 2, s0, w0)
        wait_write(nstage - 1, s1, w1)

    return k(ps_flat, table)


def kernel(samples, table):
    s32 = samples.astype(jnp.int32)
    sp = jnp.pad(s32, ((0, 0), (0, SEQ_PAD - SEQ)))
    ps = _tc_permute(sp)
    out = _sc_gather(ps.reshape(BATCH * SEQ), table)
    return out.reshape(BATCH, SEQ, EMBED)
